# baseline (speedup 1.0000x reference)
import functools

import jax
import jax.numpy as jnp
from jax.experimental import pallas as pl
from jax.experimental.pallas import tpu as pltpu

NTOK = 2048
H = 2048
DFF = 5632
NSLOT = 4

MT = 256
FT = 256
NF = DFF // FT
NM = NTOK // MT
AFT = 256
NAF = 512 // AFT


def _silu(g):
    return g * jax.nn.sigmoid(g)


def _dot_nt(a, b):
    return jax.lax.dot_general(
        a, b, (((1,), (1,)), ((), ())), preferred_element_type=jnp.float32)


def _adapter_body(ti_ref, scales_ref, x_ref,
                  rg_ref, ru_ref, rd_ref, fg_ref, fu_ref, fd_ref, out_ref,
                  rg_bf, ru_bf, rd_bf, fg_bf, fu_bf, fd_bf):
    s = pl.program_id(0)
    f = pl.program_id(1)
    m = pl.program_id(2)

    @pl.when(m == 0)
    def _():
        rg_bf[...] = rg_ref[0].astype(jnp.bfloat16)
        ru_bf[...] = ru_ref[0].astype(jnp.bfloat16)
        rd_bf[...] = rd_ref[0].astype(jnp.bfloat16)
        fg_bf[...] = fg_ref[0].astype(jnp.bfloat16)
        fu_bf[...] = fu_ref[0].astype(jnp.bfloat16)
        fd_bf[...] = fd_ref[0].astype(jnp.bfloat16)

    xm = x_ref[pl.ds(m * MT, MT), :]
    mask = (ti_ref[pl.ds(m * MT, MT)] == s).astype(jnp.float32)[:, None]
    rs = scales_ref[s, 0]
    fs = scales_ref[s, 1]

    hr = _silu(_dot_nt(xm, rg_bf[...])) * _dot_nt(xm, ru_bf[...]) * (mask * rs)
    contrib = jax.lax.dot_general(
        hr.astype(jnp.bfloat16), rd_bf[...],
        (((1,), (1,)), ((), ())), preferred_element_type=jnp.float32)
    hf = _silu(_dot_nt(xm, fg_bf[...])) * _dot_nt(xm, fu_bf[...]) * (mask * fs)
    contrib += jax.lax.dot_general(
        hf.astype(jnp.bfloat16), fd_bf[...],
        (((1,), (1,)), ((), ())), preferred_element_type=jnp.float32)

    first = (s == 0) & (f == 0)

    @pl.when(first)
    def _():
        out_ref[pl.ds(m * MT, MT), :] = contrib

    @pl.when(jnp.logical_not(first))
    def _():
        out_ref[pl.ds(m * MT, MT), :] += contrib


def _base_body(x_ref, gw_ref, uw_ref, dw_ref, add_ref, out_ref,
               gw_bf, uw_bf, dw_bf):
    f = pl.program_id(0)
    m = pl.program_id(1)

    @pl.when(m == 0)
    def _():
        gw_bf[...] = gw_ref[...].astype(jnp.bfloat16)
        uw_bf[...] = uw_ref[...].astype(jnp.bfloat16)
        dw_bf[...] = dw_ref[...].astype(jnp.bfloat16)

    xm = x_ref[pl.ds(m * MT, MT), :]
    h = _silu(_dot_nt(xm, gw_bf[...])) * _dot_nt(xm, uw_bf[...])
    contrib = jax.lax.dot_general(
        h.astype(jnp.bfloat16), dw_bf[...],
        (((1,), (1,)), ((), ())), preferred_element_type=jnp.float32)

    @pl.when(f == 0)
    def _():
        out_ref[pl.ds(m * MT, MT), :] = add_ref[pl.ds(m * MT, MT), :] + contrib

    @pl.when(f != 0)
    def _():
        out_ref[pl.ds(m * MT, MT), :] += contrib


def kernel(x, token_indices, gate_w, up_w, down_w, retain_gate, retain_up,
           retain_down, forget_gate, forget_up, forget_down, scales):
    x_bf = x.astype(jnp.bfloat16)
    ti = token_indices.astype(jnp.int32)
    full_bf = pl.BlockSpec((NTOK, H), lambda *_: (0, 0))
    full_f32 = pl.BlockSpec((NTOK, H), lambda *_: (0, 0))

    adapter_sum = pl.pallas_call(
        _adapter_body,
        grid=(NSLOT, NAF, NM),
        in_specs=[
            pl.BlockSpec((NTOK,), lambda s, f, m: (0,)),
            pl.BlockSpec(memory_space=pltpu.SMEM),
            full_bf,
            pl.BlockSpec((1, AFT, H), lambda s, f, m: (s, f, 0)),
            pl.BlockSpec((1, AFT, H), lambda s, f, m: (s, f, 0)),
            pl.BlockSpec((1, H, AFT), lambda s, f, m: (s, 0, f)),
            pl.BlockSpec((1, AFT, H), lambda s, f, m: (s, f, 0)),
            pl.BlockSpec((1, AFT, H), lambda s, f, m: (s, f, 0)),
            pl.BlockSpec((1, H, AFT), lambda s, f, m: (s, 0, f)),
        ],
        out_specs=full_f32,
        out_shape=jax.ShapeDtypeStruct((NTOK, H), jnp.float32),
        scratch_shapes=[
            pltpu.VMEM((AFT, H), jnp.bfloat16),
            pltpu.VMEM((AFT, H), jnp.bfloat16),
            pltpu.VMEM((H, AFT), jnp.bfloat16),
            pltpu.VMEM((AFT, H), jnp.bfloat16),
            pltpu.VMEM((AFT, H), jnp.bfloat16),
            pltpu.VMEM((H, AFT), jnp.bfloat16),
        ],
        compiler_params=pltpu.CompilerParams(
            dimension_semantics=("arbitrary", "arbitrary", "arbitrary")),
    )(ti, scales, x_bf,
      retain_gate, retain_up, retain_down,
      forget_gate, forget_up, forget_down)

    out = pl.pallas_call(
        _base_body,
        grid=(NF, NM),
        in_specs=[
            full_bf,
            pl.BlockSpec((FT, H), lambda f, m: (f, 0)),
            pl.BlockSpec((FT, H), lambda f, m: (f, 0)),
            pl.BlockSpec((H, FT), lambda f, m: (0, f)),
            full_f32,
        ],
        out_specs=full_f32,
        out_shape=jax.ShapeDtypeStruct((NTOK, H), jnp.float32),
        scratch_shapes=[
            pltpu.VMEM((FT, H), jnp.bfloat16),
            pltpu.VMEM((FT, H), jnp.bfloat16),
            pltpu.VMEM((H, FT), jnp.bfloat16),
        ],
        compiler_params=pltpu.CompilerParams(
            dimension_semantics=("arbitrary", "arbitrary")),
    )(x_bf, gate_w, up_w, down_w, adapter_sum)

    return out
